# baseline (speedup 1.0000x reference)
import jax
import jax.numpy as jnp
from jax.experimental import pallas as pl
from jax.experimental.pallas import tpu as pltpu

_NEG_INF = float("-inf")


def _silu(x):
    return x * (0.5 * jnp.tanh(0.5 * x) + 0.5)


def _dotg(a, b, dims):
    return jax.lax.dot_general(a, b, (dims, ((), ())),
                               preferred_element_type=jnp.float32)


def kernel(x, w_up1, w_up2, s_up2, b_up2, w_c1, s_c1, b_c1, w_c2, s_c2, b_c2,
           w_mb, s_mb, b_mb, w_c3, s_c3, b_c3, s_n, b_n, w_cb, s_cb, b_cb,
           w_cu, s_cu, b_cu):
    N, C, H, W = x.shape
    co = w_up2.shape[1]
    co2 = w_cu.shape[1]
    HW = H * W

    x3 = x.reshape(N, C, HW).astype(jnp.bfloat16)
    wc1r = w_c1.reshape(3, 3 * co, co)
    wmb4 = w_mb.reshape(4, co, co)
    wc3r = w_c3.reshape(3, 3 * co, co)

    def body(x_ref, wup1_ref, wup2_ref, su2_ref, bu2_ref, wc1_ref, sc1_ref,
             bc1_ref, wc2_ref, sc2_ref, bc2_ref, wmb_ref, smb_ref, bmb_ref,
             wc3_ref, sc3_ref, bc3_ref, sn_ref, bn_ref, wcb_ref, scb_ref,
             bcb_ref, wcu_ref, scu_ref, bcu_ref,
             p5_ref, xu_ref, scat_ref):
        wq = jax.lax.broadcasted_iota(jnp.int32, (HW + 2 * W, co), 0) % W
        m_w0 = wq == 0
        m_wl = wq == W - 1
        wqp = wq[:HW]

        def conv3x3(tin, w_ref, s_ref, b_ref):
            z = jnp.zeros((W, co), jnp.float32)
            bv = jnp.concatenate([z, tin, z], axis=0)
            s0 = jnp.where(m_w0, 0.0, pltpu.roll(bv, 1, 0))
            s2 = jnp.where(m_wl, 0.0, pltpu.roll(bv, HW + 2 * W - 1, 0))
            scat_ref[:, 0:co] = s0
            scat_ref[:, co:2 * co] = bv
            scat_ref[:, 2 * co:3 * co] = s2
            acc = _dotg(scat_ref[0:HW, :], w_ref[0], ((1,), (0,)))
            acc = acc + _dotg(scat_ref[W:W + HW, :], w_ref[1], ((1,), (0,)))
            acc = acc + _dotg(scat_ref[2 * W:2 * W + HW, :], w_ref[2],
                              ((1,), (0,)))
            return _silu(acc * s_ref[...] + b_ref[...])

        minf4 = jnp.full((4 * W, co), _NEG_INF, jnp.float32)

        def sh_dn(v, k):
            return jnp.concatenate([minf4[:k * W], v[:HW - k * W]], axis=0)

        def sh_up(v, k):
            return jnp.concatenate([v[k * W:], minf4[:k * W]], axis=0)

        def hpass5(u):
            hm = jnp.maximum(jnp.maximum(u, sh_dn(u, 1)), sh_dn(u, 2))
            return jnp.maximum(jnp.maximum(hm, sh_up(u, 1)), sh_up(u, 2))

        def sppf_pools(v):
            u5 = jnp.maximum(
                jnp.maximum(v, jnp.where(wqp == 0, _NEG_INF,
                                         pltpu.roll(v, 1, 0))),
                jnp.where(wqp <= 1, _NEG_INF, pltpu.roll(v, 2, 0)))
            u5 = jnp.maximum(
                jnp.maximum(u5, jnp.where(wqp == W - 1, _NEG_INF,
                                          pltpu.roll(v, HW - 1, 0))),
                jnp.where(wqp >= W - 2, _NEG_INF, pltpu.roll(v, HW - 2, 0)))
            u9 = jnp.maximum(
                jnp.maximum(u5, jnp.where(wqp <= 1, _NEG_INF,
                                          pltpu.roll(u5, 2, 0))),
                jnp.where(wqp >= W - 2, _NEG_INF, pltpu.roll(u5, HW - 2, 0)))
            u13 = jnp.maximum(
                jnp.maximum(u9, jnp.where(wqp <= 3, _NEG_INF,
                                          pltpu.roll(u5, 4, 0))),
                jnp.where(wqp >= W - 4, _NEG_INF, pltpu.roll(u5, HW - 4, 0)))
            p5 = hpass5(u5)
            v9 = hpass5(u9)
            p9 = jnp.maximum(jnp.maximum(v9, sh_dn(v9, 2)), sh_up(v9, 2))
            v13 = hpass5(u13)
            p13 = jnp.maximum(
                jnp.maximum(jnp.maximum(v13, sh_dn(v13, 2)), sh_up(v13, 2)),
                jnp.maximum(sh_dn(v13, 4), sh_up(v13, 4)))
            return p5, p9, p13

        xc = x_ref[0]
        y2 = _dotg(xc, wup1_ref[...], ((0,), (0,)))
        t0 = _silu(_dotg(xc, wup2_ref[...], ((0,), (0,)))
                         * su2_ref[...] + bu2_ref[...])

        t1 = conv3x3(t0, wc1_ref, sc1_ref, bc1_ref)
        t2 = _silu(_dotg(t1, wc2_ref[...], ((1,), (0,)))
                         * sc2_ref[...] + bc2_ref[...])

        acc = _dotg(t2, wmb_ref[0], ((1,), (0,)))
        mp5, mp9, mp13 = sppf_pools(t2)
        acc = acc + _dotg(mp5, wmb_ref[1], ((1,), (0,)))
        acc = acc + _dotg(mp9, wmb_ref[2], ((1,), (0,)))
        acc = acc + _dotg(mp13, wmb_ref[3], ((1,), (0,)))
        y1 = _silu(acc * smb_ref[...] + bmb_ref[...])

        y1 = conv3x3(y1, wc3_ref, sc3_ref, bc3_ref)

        z1 = _silu(y1 * sn_ref[:, :co] + bn_ref[:, :co])
        z2 = _silu(y2 * sn_ref[:, co:] + bn_ref[:, co:])
        zc = jnp.concatenate([z1, z2], axis=1)
        accT = _dotg(wcb_ref[...], zc, ((0,), (1,)))
        p5T = _silu(accT * scb_ref[...].T + bcb_ref[...].T)
        p5_ref[0] = p5T
        xuT = _dotg(wcu_ref[...], p5T, ((0,), (0,)))
        xu_ref[0] = _silu(xuT * scu_ref[...].T + bcu_ref[...].T)

    def full(shape):
        nd = len(shape)
        return pl.BlockSpec(shape, lambda n, _nd=nd: (0,) * _nd)

    p5t, xut = pl.pallas_call(
        body,
        grid=(N,),
        out_shape=(jax.ShapeDtypeStruct((N, co, HW), jnp.float32),
                   jax.ShapeDtypeStruct((N, co2, HW), jnp.float32)),
        in_specs=[pl.BlockSpec((1, C, HW), lambda n: (n, 0, 0)),
                  full((C, co)), full((C, co)), full((1, co)), full((1, co)),
                  full((3, 3 * co, co)), full((1, co)), full((1, co)),
                  full((co, co)), full((1, co)), full((1, co)),
                  full((4, co, co)), full((1, co)), full((1, co)),
                  full((3, 3 * co, co)), full((1, co)), full((1, co)),
                  full((1, 2 * co)), full((1, 2 * co)),
                  full((2 * co, co)), full((1, co)), full((1, co)),
                  full((co, co2)), full((1, co2)), full((1, co2))],
        out_specs=(pl.BlockSpec((1, co, HW), lambda n: (n, 0, 0)),
                   pl.BlockSpec((1, co2, HW), lambda n: (n, 0, 0))),
        scratch_shapes=[pltpu.VMEM((HW + 2 * W, 3 * co), jnp.float32)],
        compiler_params=pltpu.CompilerParams(
            dimension_semantics=("arbitrary",),
            vmem_limit_bytes=100 * 1024 * 1024),
    )(x3, w_up1, w_up2, s_up2, b_up2, wc1r, s_c1, b_c1, w_c2, s_c2, b_c2,
      wmb4, s_mb, b_mb, wc3r, s_c3, b_c3, s_n, b_n, w_cb, s_cb, b_cb,
      w_cu, s_cu, b_cu)

    p5 = p5t.reshape(N, co, H, W)
    xu = xut.reshape(N, co2, H, W)
    xu = jnp.broadcast_to(xu[:, :, :, None, :, None],
                          (N, co2, H, 2, W, 2)).reshape(N, co2, 2 * H, 2 * W)
    return (p5, xu)


# speedup vs baseline: 1.0199x; 1.0199x over previous
import jax
import jax.numpy as jnp
from jax.experimental import pallas as pl
from jax.experimental.pallas import tpu as pltpu

_NEG_INF = float("-inf")


def _silu(x):
    return x * (0.5 * jnp.tanh(0.5 * x) + 0.5)


def _dotg(a, b, dims):
    return jax.lax.dot_general(a, b, (dims, ((), ())),
                               preferred_element_type=jnp.float32)


def kernel(x, w_up1, w_up2, s_up2, b_up2, w_c1, s_c1, b_c1, w_c2, s_c2, b_c2,
           w_mb, s_mb, b_mb, w_c3, s_c3, b_c3, s_n, b_n, w_cb, s_cb, b_cb,
           w_cu, s_cu, b_cu):
    N, C, H, W = x.shape
    co = w_up2.shape[1]
    co2 = w_cu.shape[1]
    HW = H * W

    x3 = x.reshape(N, C, HW)
    wc1r = w_c1.reshape(3, 3 * co, co)
    wmb4 = w_mb.reshape(4, co, co)
    wc3r = w_c3.reshape(3, 3 * co, co)

    def body(x_ref, wup1_ref, wup2_ref, su2_ref, bu2_ref, wc1_ref, sc1_ref,
             bc1_ref, wc2_ref, sc2_ref, bc2_ref, wmb_ref, smb_ref, bmb_ref,
             wc3_ref, sc3_ref, bc3_ref, sn_ref, bn_ref, wcb_ref, scb_ref,
             bcb_ref, wcu_ref, scu_ref, bcu_ref,
             p5_ref, xu_ref, scat_ref):
        wq = jax.lax.broadcasted_iota(jnp.int32, (HW + 2 * W, co), 0) % W
        m_w0 = wq == 0
        m_wl = wq == W - 1
        wqp = wq[:HW]

        def conv3x3(tin, w_ref, s_ref, b_ref):
            z = jnp.zeros((W, co), jnp.float32)
            bv = jnp.concatenate([z, tin, z], axis=0)
            s0 = jnp.where(m_w0, 0.0, pltpu.roll(bv, 1, 0))
            s2 = jnp.where(m_wl, 0.0, pltpu.roll(bv, HW + 2 * W - 1, 0))
            scat_ref[:, 0:co] = s0
            scat_ref[:, co:2 * co] = bv
            scat_ref[:, 2 * co:3 * co] = s2
            acc = _dotg(scat_ref[0:HW, :], w_ref[0], ((1,), (0,)))
            acc = acc + _dotg(scat_ref[W:W + HW, :], w_ref[1], ((1,), (0,)))
            acc = acc + _dotg(scat_ref[2 * W:2 * W + HW, :], w_ref[2],
                              ((1,), (0,)))
            return _silu(acc * s_ref[...] + b_ref[...])

        minf4 = jnp.full((4 * W, co), _NEG_INF, jnp.float32)

        def sh_dn(v, k):
            return jnp.concatenate([minf4[:k * W], v[:HW - k * W]], axis=0)

        def sh_up(v, k):
            return jnp.concatenate([v[k * W:], minf4[:k * W]], axis=0)

        def hpass5(u):
            hm = jnp.maximum(jnp.maximum(u, sh_dn(u, 1)), sh_dn(u, 2))
            return jnp.maximum(jnp.maximum(hm, sh_up(u, 1)), sh_up(u, 2))

        def sppf_pools(v):
            u5 = jnp.maximum(
                jnp.maximum(v, jnp.where(wqp == 0, _NEG_INF,
                                         pltpu.roll(v, 1, 0))),
                jnp.where(wqp <= 1, _NEG_INF, pltpu.roll(v, 2, 0)))
            u5 = jnp.maximum(
                jnp.maximum(u5, jnp.where(wqp == W - 1, _NEG_INF,
                                          pltpu.roll(v, HW - 1, 0))),
                jnp.where(wqp >= W - 2, _NEG_INF, pltpu.roll(v, HW - 2, 0)))
            u9 = jnp.maximum(
                jnp.maximum(u5, jnp.where(wqp <= 1, _NEG_INF,
                                          pltpu.roll(u5, 2, 0))),
                jnp.where(wqp >= W - 2, _NEG_INF, pltpu.roll(u5, HW - 2, 0)))
            u13 = jnp.maximum(
                jnp.maximum(u9, jnp.where(wqp <= 3, _NEG_INF,
                                          pltpu.roll(u5, 4, 0))),
                jnp.where(wqp >= W - 4, _NEG_INF, pltpu.roll(u5, HW - 4, 0)))
            p5 = hpass5(u5)
            v9 = hpass5(u9)
            p9 = jnp.maximum(jnp.maximum(v9, sh_dn(v9, 2)), sh_up(v9, 2))
            v13 = hpass5(u13)
            p13 = jnp.maximum(
                jnp.maximum(jnp.maximum(v13, sh_dn(v13, 2)), sh_up(v13, 2)),
                jnp.maximum(sh_dn(v13, 4), sh_up(v13, 4)))
            return p5, p9, p13

        xc = x_ref[0]
        y2 = _dotg(xc, wup1_ref[...], ((0,), (0,)))
        t0 = _silu(_dotg(xc, wup2_ref[...], ((0,), (0,)))
                         * su2_ref[...] + bu2_ref[...])

        t1 = conv3x3(t0, wc1_ref, sc1_ref, bc1_ref)
        t2 = _silu(_dotg(t1, wc2_ref[...], ((1,), (0,)))
                         * sc2_ref[...] + bc2_ref[...])

        acc = _dotg(t2, wmb_ref[0], ((1,), (0,)))
        mp5, mp9, mp13 = sppf_pools(t2)
        acc = acc + _dotg(mp5, wmb_ref[1], ((1,), (0,)))
        acc = acc + _dotg(mp9, wmb_ref[2], ((1,), (0,)))
        acc = acc + _dotg(mp13, wmb_ref[3], ((1,), (0,)))
        y1 = _silu(acc * smb_ref[...] + bmb_ref[...])

        y1 = conv3x3(y1, wc3_ref, sc3_ref, bc3_ref)

        z1 = _silu(y1 * sn_ref[:, :co] + bn_ref[:, :co])
        z2 = _silu(y2 * sn_ref[:, co:] + bn_ref[:, co:])
        zc = jnp.concatenate([z1, z2], axis=1)
        accT = _dotg(wcb_ref[...], zc, ((0,), (1,)))
        p5T = _silu(accT * scb_ref[...].T + bcb_ref[...].T)
        p5_ref[0] = p5T
        xuT = _dotg(wcu_ref[...], p5T, ((0,), (0,)))
        xu_ref[0] = _silu(xuT * scu_ref[...].T + bcu_ref[...].T)

    def full(shape):
        nd = len(shape)
        return pl.BlockSpec(shape, lambda n, _nd=nd: (0,) * _nd)

    p5t, xut = pl.pallas_call(
        body,
        grid=(N,),
        out_shape=(jax.ShapeDtypeStruct((N, co, HW), jnp.float32),
                   jax.ShapeDtypeStruct((N, co2, HW), jnp.float32)),
        in_specs=[pl.BlockSpec((1, C, HW), lambda n: (n, 0, 0)),
                  full((C, co)), full((C, co)), full((1, co)), full((1, co)),
                  full((3, 3 * co, co)), full((1, co)), full((1, co)),
                  full((co, co)), full((1, co)), full((1, co)),
                  full((4, co, co)), full((1, co)), full((1, co)),
                  full((3, 3 * co, co)), full((1, co)), full((1, co)),
                  full((1, 2 * co)), full((1, 2 * co)),
                  full((2 * co, co)), full((1, co)), full((1, co)),
                  full((co, co2)), full((1, co2)), full((1, co2))],
        out_specs=(pl.BlockSpec((1, co, HW), lambda n: (n, 0, 0)),
                   pl.BlockSpec((1, co2, HW), lambda n: (n, 0, 0))),
        scratch_shapes=[pltpu.VMEM((HW + 2 * W, 3 * co), jnp.float32)],
        compiler_params=pltpu.CompilerParams(
            dimension_semantics=("arbitrary",),
            vmem_limit_bytes=100 * 1024 * 1024),
    )(x3, w_up1, w_up2, s_up2, b_up2, wc1r, s_c1, b_c1, w_c2, s_c2, b_c2,
      wmb4, s_mb, b_mb, wc3r, s_c3, b_c3, s_n, b_n, w_cb, s_cb, b_cb,
      w_cu, s_cu, b_cu)

    p5 = p5t.reshape(N, co, H, W)
    xu = xut.reshape(N, co2, H, W)
    xu = jnp.broadcast_to(xu[:, :, :, None, :, None],
                          (N, co2, H, 2, W, 2)).reshape(N, co2, 2 * H, 2 * W)
    return (p5, xu)


# speedup vs baseline: 1.2368x; 1.2127x over previous
import jax
import jax.numpy as jnp
from jax.experimental import pallas as pl
from jax.experimental.pallas import tpu as pltpu

_NEG_INF = float("-inf")


def _silu(x):
    return x * (0.5 * jnp.tanh(0.5 * x) + 0.5)


def _dotg(a, b, dims):
    return jax.lax.dot_general(a, b, (dims, ((), ())),
                               preferred_element_type=jnp.float32)


def kernel(x, w_up1, w_up2, s_up2, b_up2, w_c1, s_c1, b_c1, w_c2, s_c2, b_c2,
           w_mb, s_mb, b_mb, w_c3, s_c3, b_c3, s_n, b_n, w_cb, s_cb, b_cb,
           w_cu, s_cu, b_cu):
    N, C, H, W = x.shape
    co = w_up2.shape[1]
    co2 = w_cu.shape[1]
    HW = H * W
    G = 2

    x3 = x.reshape(N, C, HW)
    wc1r = w_c1.reshape(3, 3 * co, co)
    wmb4 = w_mb.reshape(4, co, co)
    wc3r = w_c3.reshape(3, 3 * co, co)

    def body(x_ref, wup1_ref, wup2_ref, su2_ref, bu2_ref, wc1_ref, sc1_ref,
             bc1_ref, wc2_ref, sc2_ref, bc2_ref, wmb_ref, smb_ref, bmb_ref,
             wc3_ref, sc3_ref, bc3_ref, sn_ref, bn_ref, wcb_ref, scb_ref,
             bcb_ref, wcu_ref, scu_ref, bcu_ref,
             p5_ref, xu_ref, scat0_ref, scat1_ref):
        scats = [scat0_ref, scat1_ref]
        wq = jax.lax.broadcasted_iota(jnp.int32, (HW + 2 * W, co), 0) % W
        m_w0 = wq == 0
        m_wl = wq == W - 1
        wqp = wq[:HW]
        minf4 = jnp.full((4 * W, co), _NEG_INF, jnp.float32)

        def sh_dn(v, k):
            return jnp.concatenate([minf4[:k * W], v[:HW - k * W]], axis=0)

        def sh_up(v, k):
            return jnp.concatenate([v[k * W:], minf4[:k * W]], axis=0)

        def hpass5(u):
            hm = jnp.maximum(jnp.maximum(u, sh_dn(u, 1)), sh_dn(u, 2))
            return jnp.maximum(jnp.maximum(hm, sh_up(u, 1)), sh_up(u, 2))

        def conv_store(tin, scat_ref):
            z = jnp.zeros((W, co), jnp.float32)
            bv = jnp.concatenate([z, tin, z], axis=0)
            s0 = jnp.where(m_w0, 0.0, pltpu.roll(bv, 1, 0))
            s2 = jnp.where(m_wl, 0.0, pltpu.roll(bv, HW + 2 * W - 1, 0))
            scat_ref[:, 0:co] = s0
            scat_ref[:, co:2 * co] = bv
            scat_ref[:, 2 * co:3 * co] = s2

        def conv_dots(scat_ref, w_ref, s_ref, b_ref):
            acc = _dotg(scat_ref[0:HW, :], w_ref[0], ((1,), (0,)))
            acc = acc + _dotg(scat_ref[W:W + HW, :], w_ref[1], ((1,), (0,)))
            acc = acc + _dotg(scat_ref[2 * W:2 * W + HW, :], w_ref[2],
                              ((1,), (0,)))
            return _silu(acc * s_ref[...] + b_ref[...])

        def sppf_pools(v):
            u5 = jnp.maximum(
                jnp.maximum(v, jnp.where(wqp == 0, _NEG_INF,
                                         pltpu.roll(v, 1, 0))),
                jnp.where(wqp <= 1, _NEG_INF, pltpu.roll(v, 2, 0)))
            u5 = jnp.maximum(
                jnp.maximum(u5, jnp.where(wqp == W - 1, _NEG_INF,
                                          pltpu.roll(v, HW - 1, 0))),
                jnp.where(wqp >= W - 2, _NEG_INF, pltpu.roll(v, HW - 2, 0)))
            u9 = jnp.maximum(
                jnp.maximum(u5, jnp.where(wqp <= 1, _NEG_INF,
                                          pltpu.roll(u5, 2, 0))),
                jnp.where(wqp >= W - 2, _NEG_INF, pltpu.roll(u5, HW - 2, 0)))
            u13 = jnp.maximum(
                jnp.maximum(u9, jnp.where(wqp <= 3, _NEG_INF,
                                          pltpu.roll(u5, 4, 0))),
                jnp.where(wqp >= W - 4, _NEG_INF, pltpu.roll(u5, HW - 4, 0)))
            p5 = hpass5(u5)
            v9 = hpass5(u9)
            p9 = jnp.maximum(jnp.maximum(v9, sh_dn(v9, 2)), sh_up(v9, 2))
            v13 = hpass5(u13)
            p13 = jnp.maximum(
                jnp.maximum(jnp.maximum(v13, sh_dn(v13, 2)), sh_up(v13, 2)),
                jnp.maximum(sh_dn(v13, 4), sh_up(v13, 4)))
            return p5, p9, p13

        y2 = [None] * G
        t0 = [None] * G
        t1 = [None] * G
        t2 = [None] * G
        y1 = [None] * G
        ps = [None] * G
        for g in range(G):
            xc = x_ref[g]
            y2[g] = _dotg(xc, wup1_ref[...], ((0,), (0,)))
            t0[g] = _silu(_dotg(xc, wup2_ref[...], ((0,), (0,)))
                          * su2_ref[...] + bu2_ref[...])
        for g in range(G):
            conv_store(t0[g], scats[g])
        for g in range(G):
            t1[g] = conv_dots(scats[g], wc1_ref, sc1_ref, bc1_ref)
        for g in range(G):
            t2[g] = _silu(_dotg(t1[g], wc2_ref[...], ((1,), (0,)))
                          * sc2_ref[...] + bc2_ref[...])
        for g in range(G):
            ps[g] = sppf_pools(t2[g])
        for g in range(G):
            mp5, mp9, mp13 = ps[g]
            acc = _dotg(t2[g], wmb_ref[0], ((1,), (0,)))
            acc = acc + _dotg(mp5, wmb_ref[1], ((1,), (0,)))
            acc = acc + _dotg(mp9, wmb_ref[2], ((1,), (0,)))
            acc = acc + _dotg(mp13, wmb_ref[3], ((1,), (0,)))
            y1[g] = _silu(acc * smb_ref[...] + bmb_ref[...])
        for g in range(G):
            conv_store(y1[g], scats[g])
        for g in range(G):
            y1[g] = conv_dots(scats[g], wc3_ref, sc3_ref, bc3_ref)
        for g in range(G):
            z1 = _silu(y1[g] * sn_ref[:, :co] + bn_ref[:, :co])
            z2 = _silu(y2[g] * sn_ref[:, co:] + bn_ref[:, co:])
            zc = jnp.concatenate([z1, z2], axis=1)
            accT = _dotg(wcb_ref[...], zc, ((0,), (1,)))
            p5T = _silu(accT * scb_ref[...].T + bcb_ref[...].T)
            p5_ref[g] = p5T
            xuT = _dotg(wcu_ref[...], p5T, ((0,), (0,)))
            xu_ref[g] = _silu(xuT * scu_ref[...].T + bcu_ref[...].T)

    def full(shape):
        nd = len(shape)
        return pl.BlockSpec(shape, lambda n, _nd=nd: (0,) * _nd)

    p5t, xut = pl.pallas_call(
        body,
        grid=(N // G,),
        out_shape=(jax.ShapeDtypeStruct((N, co, HW), jnp.float32),
                   jax.ShapeDtypeStruct((N, co2, HW), jnp.float32)),
        in_specs=[pl.BlockSpec((G, C, HW), lambda n: (n, 0, 0)),
                  full((C, co)), full((C, co)), full((1, co)), full((1, co)),
                  full((3, 3 * co, co)), full((1, co)), full((1, co)),
                  full((co, co)), full((1, co)), full((1, co)),
                  full((4, co, co)), full((1, co)), full((1, co)),
                  full((3, 3 * co, co)), full((1, co)), full((1, co)),
                  full((1, 2 * co)), full((1, 2 * co)),
                  full((2 * co, co)), full((1, co)), full((1, co)),
                  full((co, co2)), full((1, co2)), full((1, co2))],
        out_specs=(pl.BlockSpec((G, co, HW), lambda n: (n, 0, 0)),
                   pl.BlockSpec((G, co2, HW), lambda n: (n, 0, 0))),
        scratch_shapes=[pltpu.VMEM((HW + 2 * W, 3 * co), jnp.float32),
                        pltpu.VMEM((HW + 2 * W, 3 * co), jnp.float32)],
        compiler_params=pltpu.CompilerParams(
            dimension_semantics=("arbitrary",),
            vmem_limit_bytes=100 * 1024 * 1024),
    )(x3, w_up1, w_up2, s_up2, b_up2, wc1r, s_c1, b_c1, w_c2, s_c2, b_c2,
      wmb4, s_mb, b_mb, wc3r, s_c3, b_c3, s_n, b_n, w_cb, s_cb, b_cb,
      w_cu, s_cu, b_cu)

    p5 = p5t.reshape(N, co, H, W)
    xu = xut.reshape(N, co2, H, W)
    xu = jnp.broadcast_to(xu[:, :, :, None, :, None],
                          (N, co2, H, 2, W, 2)).reshape(N, co2, 2 * H, 2 * W)
    return (p5, xu)


# speedup vs baseline: 1.2458x; 1.0073x over previous
import jax
import jax.numpy as jnp
from jax.experimental import pallas as pl
from jax.experimental.pallas import tpu as pltpu

_NEG_INF = float("-inf")


def _silu(x):
    return x * (0.5 * jnp.tanh(0.5 * x) + 0.5)


def _dotg(a, b, dims):
    return jax.lax.dot_general(a, b, (dims, ((), ())),
                               preferred_element_type=jnp.float32)


def kernel(x, w_up1, w_up2, s_up2, b_up2, w_c1, s_c1, b_c1, w_c2, s_c2, b_c2,
           w_mb, s_mb, b_mb, w_c3, s_c3, b_c3, s_n, b_n, w_cb, s_cb, b_cb,
           w_cu, s_cu, b_cu):
    N, C, H, W = x.shape
    co = w_up2.shape[1]
    co2 = w_cu.shape[1]
    HW = H * W
    G = 4

    x3 = x.reshape(N, C, HW)
    wc1r = w_c1.reshape(3, 3 * co, co)
    wmb4 = w_mb.reshape(4, co, co)
    wc3r = w_c3.reshape(3, 3 * co, co)

    def body(x_ref, wup1_ref, wup2_ref, su2_ref, bu2_ref, wc1_ref, sc1_ref,
             bc1_ref, wc2_ref, sc2_ref, bc2_ref, wmb_ref, smb_ref, bmb_ref,
             wc3_ref, sc3_ref, bc3_ref, sn_ref, bn_ref, wcb_ref, scb_ref,
             bcb_ref, wcu_ref, scu_ref, bcu_ref,
             p5_ref, xu_ref, scat0_ref, scat1_ref, scat2_ref, scat3_ref):
        scats = [scat0_ref, scat1_ref, scat2_ref, scat3_ref]
        wq = jax.lax.broadcasted_iota(jnp.int32, (HW + 2 * W, co), 0) % W
        m_w0 = wq == 0
        m_wl = wq == W - 1
        wqp = wq[:HW]
        minf4 = jnp.full((4 * W, co), _NEG_INF, jnp.float32)

        def sh_dn(v, k):
            return jnp.concatenate([minf4[:k * W], v[:HW - k * W]], axis=0)

        def sh_up(v, k):
            return jnp.concatenate([v[k * W:], minf4[:k * W]], axis=0)

        def hpass5(u):
            hm = jnp.maximum(jnp.maximum(u, sh_dn(u, 1)), sh_dn(u, 2))
            return jnp.maximum(jnp.maximum(hm, sh_up(u, 1)), sh_up(u, 2))

        def conv_store(tin, scat_ref):
            z = jnp.zeros((W, co), jnp.float32)
            bv = jnp.concatenate([z, tin, z], axis=0)
            s0 = jnp.where(m_w0, 0.0, pltpu.roll(bv, 1, 0))
            s2 = jnp.where(m_wl, 0.0, pltpu.roll(bv, HW + 2 * W - 1, 0))
            scat_ref[:, 0:co] = s0
            scat_ref[:, co:2 * co] = bv
            scat_ref[:, 2 * co:3 * co] = s2

        def conv_dots(scat_ref, w_ref, s_ref, b_ref):
            acc = _dotg(scat_ref[0:HW, :], w_ref[0], ((1,), (0,)))
            acc = acc + _dotg(scat_ref[W:W + HW, :], w_ref[1], ((1,), (0,)))
            acc = acc + _dotg(scat_ref[2 * W:2 * W + HW, :], w_ref[2],
                              ((1,), (0,)))
            return _silu(acc * s_ref[...] + b_ref[...])

        def sppf_pools(v):
            u5 = jnp.maximum(
                jnp.maximum(v, jnp.where(wqp == 0, _NEG_INF,
                                         pltpu.roll(v, 1, 0))),
                jnp.where(wqp <= 1, _NEG_INF, pltpu.roll(v, 2, 0)))
            u5 = jnp.maximum(
                jnp.maximum(u5, jnp.where(wqp == W - 1, _NEG_INF,
                                          pltpu.roll(v, HW - 1, 0))),
                jnp.where(wqp >= W - 2, _NEG_INF, pltpu.roll(v, HW - 2, 0)))
            u9 = jnp.maximum(
                jnp.maximum(u5, jnp.where(wqp <= 1, _NEG_INF,
                                          pltpu.roll(u5, 2, 0))),
                jnp.where(wqp >= W - 2, _NEG_INF, pltpu.roll(u5, HW - 2, 0)))
            u13 = jnp.maximum(
                jnp.maximum(u9, jnp.where(wqp <= 3, _NEG_INF,
                                          pltpu.roll(u5, 4, 0))),
                jnp.where(wqp >= W - 4, _NEG_INF, pltpu.roll(u5, HW - 4, 0)))
            p5 = hpass5(u5)
            v9 = hpass5(u9)
            p9 = jnp.maximum(jnp.maximum(v9, sh_dn(v9, 2)), sh_up(v9, 2))
            v13 = hpass5(u13)
            p13 = jnp.maximum(
                jnp.maximum(jnp.maximum(v13, sh_dn(v13, 2)), sh_up(v13, 2)),
                jnp.maximum(sh_dn(v13, 4), sh_up(v13, 4)))
            return p5, p9, p13

        y2 = [None] * G
        t0 = [None] * G
        t1 = [None] * G
        t2 = [None] * G
        y1 = [None] * G
        ps = [None] * G
        for g in range(G):
            xc = x_ref[g]
            y2[g] = _dotg(xc, wup1_ref[...], ((0,), (0,)))
            t0[g] = _silu(_dotg(xc, wup2_ref[...], ((0,), (0,)))
                          * su2_ref[...] + bu2_ref[...])
        for g in range(G):
            conv_store(t0[g], scats[g])
        for g in range(G):
            t1[g] = conv_dots(scats[g], wc1_ref, sc1_ref, bc1_ref)
        for g in range(G):
            t2[g] = _silu(_dotg(t1[g], wc2_ref[...], ((1,), (0,)))
                          * sc2_ref[...] + bc2_ref[...])
        for g in range(G):
            ps[g] = sppf_pools(t2[g])
        for g in range(G):
            mp5, mp9, mp13 = ps[g]
            acc = _dotg(t2[g], wmb_ref[0], ((1,), (0,)))
            acc = acc + _dotg(mp5, wmb_ref[1], ((1,), (0,)))
            acc = acc + _dotg(mp9, wmb_ref[2], ((1,), (0,)))
            acc = acc + _dotg(mp13, wmb_ref[3], ((1,), (0,)))
            y1[g] = _silu(acc * smb_ref[...] + bmb_ref[...])
        for g in range(G):
            conv_store(y1[g], scats[g])
        for g in range(G):
            y1[g] = conv_dots(scats[g], wc3_ref, sc3_ref, bc3_ref)
        for g in range(G):
            z1 = _silu(y1[g] * sn_ref[:, :co] + bn_ref[:, :co])
            z2 = _silu(y2[g] * sn_ref[:, co:] + bn_ref[:, co:])
            zc = jnp.concatenate([z1, z2], axis=1)
            accT = _dotg(wcb_ref[...], zc, ((0,), (1,)))
            p5T = _silu(accT * scb_ref[...].T + bcb_ref[...].T)
            p5_ref[g] = p5T
            xuT = _dotg(wcu_ref[...], p5T, ((0,), (0,)))
            xu_ref[g] = _silu(xuT * scu_ref[...].T + bcu_ref[...].T)

    def full(shape):
        nd = len(shape)
        return pl.BlockSpec(shape, lambda n, _nd=nd: (0,) * _nd)

    p5t, xut = pl.pallas_call(
        body,
        grid=(N // G,),
        out_shape=(jax.ShapeDtypeStruct((N, co, HW), jnp.float32),
                   jax.ShapeDtypeStruct((N, co2, HW), jnp.float32)),
        in_specs=[pl.BlockSpec((G, C, HW), lambda n: (n, 0, 0)),
                  full((C, co)), full((C, co)), full((1, co)), full((1, co)),
                  full((3, 3 * co, co)), full((1, co)), full((1, co)),
                  full((co, co)), full((1, co)), full((1, co)),
                  full((4, co, co)), full((1, co)), full((1, co)),
                  full((3, 3 * co, co)), full((1, co)), full((1, co)),
                  full((1, 2 * co)), full((1, 2 * co)),
                  full((2 * co, co)), full((1, co)), full((1, co)),
                  full((co, co2)), full((1, co2)), full((1, co2))],
        out_specs=(pl.BlockSpec((G, co, HW), lambda n: (n, 0, 0)),
                   pl.BlockSpec((G, co2, HW), lambda n: (n, 0, 0))),
        scratch_shapes=[pltpu.VMEM((HW + 2 * W, 3 * co), jnp.float32)
                        for _ in range(G)],
        compiler_params=pltpu.CompilerParams(
            dimension_semantics=("arbitrary",),
            vmem_limit_bytes=100 * 1024 * 1024),
    )(x3, w_up1, w_up2, s_up2, b_up2, wc1r, s_c1, b_c1, w_c2, s_c2, b_c2,
      wmb4, s_mb, b_mb, wc3r, s_c3, b_c3, s_n, b_n, w_cb, s_cb, b_cb,
      w_cu, s_cu, b_cu)

    p5 = p5t.reshape(N, co, H, W)
    xu = xut.reshape(N, co2, H, W)
    xu = jnp.broadcast_to(xu[:, :, :, None, :, None],
                          (N, co2, H, 2, W, 2)).reshape(N, co2, 2 * H, 2 * W)
    return (p5, xu)
